# Initial kernel scaffold; baseline (speedup 1.0000x reference)
import jax
import jax.numpy as jnp
from jax.experimental import pallas as pl


def kernel(primitives_raw, positions):
    raise NotImplementedError("write your pallas kernel here")


# speedup vs baseline: 2.8612x; 2.8612x over previous
import functools

import jax
import jax.numpy as jnp
from jax import lax
from jax.experimental import pallas as pl
from jax.experimental.pallas import tpu as pltpu

DIM = 32
N_POS = 2048
WMAX = 16
BLK = 256
TILE_R = 256
EXPM_SCALE_LOG2 = 4
EXPM_ORDER = 20


def _eye(n, dtype=jnp.float32):
    r = lax.broadcasted_iota(jnp.int32, (n, n), 0)
    c = lax.broadcasted_iota(jnp.int32, (n, n), 1)
    return (r == c).astype(dtype)


def _floor_log2(v):
    f = v.astype(jnp.float32)
    return ((lax.bitcast_convert_type(f, jnp.int32) >> 23) & 0xFF) - 127


def _prims_kernel(raw_ref, raw_t_ref, w_ref):
    eye = _eye(DIM)
    inv_scale = jnp.float32(2.0 ** -EXPM_SCALE_LOG2)
    for b in range(2):
        x = (raw_t_ref[b] - raw_ref[b]) * inv_scale
        r = eye
        for k in range(EXPM_ORDER, 0, -1):
            r = eye + jnp.dot(x, r, preferred_element_type=jnp.float32, precision=lax.Precision.HIGHEST) * (
                jnp.float32(1.0 / k))
        for _ in range(EXPM_SCALE_LOG2):
            r = jnp.dot(r, r, preferred_element_type=jnp.float32, precision=lax.Precision.HIGHEST)
        w_ref[b] = r


def _maps_kernel(pos_ref, w_ref, maps_ref):
    m = pos_ref[...] + 1
    depth = _floor_log2(m)
    w01 = jnp.concatenate([w_ref[0], w_ref[1]], axis=1).astype(jnp.bfloat16)

    ri = lax.broadcasted_iota(jnp.int32, (BLK, DIM, DIM), 1)
    ci = lax.broadcasted_iota(jnp.int32, (BLK, DIM, DIM), 2)
    maps = (ri == ci).astype(jnp.float32)

    for s in range(WMAX):
        shift = jnp.maximum(depth - 1 - s, 0)
        bit = (m >> shift) & 1
        active = depth > s
        u = jnp.dot(maps.reshape(BLK * DIM, DIM).astype(jnp.bfloat16), w01,
                    preferred_element_type=jnp.float32)
        u = u.reshape(BLK, DIM, 2 * DIM)
        sel_a = (active & (bit == 0))[:, :, None]
        sel_b = (active & (bit == 1))[:, :, None]
        maps = jnp.where(sel_a, u[:, :, :DIM],
                         jnp.where(sel_b, u[:, :, DIM:], maps))
    maps_ref[...] = maps.reshape(BLK, 1, DIM, DIM)


def _steps_kernel(pos_r_ref, pos_c_ref, out_ref):
    mc = pos_c_ref[...] + 1
    mr = pos_r_ref[...] + 1
    dc = _floor_log2(mc)
    dr = _floor_log2(mr)
    lens = jnp.maximum(jnp.max(dc), 1)
    xc = mc << (WMAX - dc)
    xr = mr << (WMAX - dr)
    xo = jnp.bitwise_xor(xr, xc)
    blen = _floor_log2(xo) + 1
    cpl = jnp.minimum(jnp.minimum(WMAX - blen, dr), dc)
    cpl = jnp.where(mr == mc, WMAX, cpl)
    cpl = jnp.minimum(cpl, lens)
    out_ref[...] = 2 * lens - 2 * cpl


def _compute_prims(raw01, raw01_t):
    return pl.pallas_call(
        _prims_kernel,
        out_shape=jax.ShapeDtypeStruct((2, DIM, DIM), jnp.float32),
    )(raw01, raw01_t)


def _compute_maps(pos_col, w):
    return pl.pallas_call(
        _maps_kernel,
        grid=(N_POS // BLK,),
        in_specs=[
            pl.BlockSpec((BLK, 1), lambda i: (i, 0)),
            pl.BlockSpec((2, DIM, DIM), lambda i: (0, 0, 0)),
        ],
        out_specs=pl.BlockSpec((BLK, 1, DIM, DIM), lambda i: (i, 0, 0, 0)),
        out_shape=jax.ShapeDtypeStruct((N_POS, 1, DIM, DIM), jnp.float32),
    )(pos_col, w)


def _compute_steps(pos_col, pos_row):
    return pl.pallas_call(
        _steps_kernel,
        grid=(N_POS // TILE_R,),
        in_specs=[
            pl.BlockSpec((TILE_R, 1), lambda i: (i, 0)),
            pl.BlockSpec((1, N_POS), lambda i: (0, 0)),
        ],
        out_specs=pl.BlockSpec((TILE_R, N_POS), lambda i: (i, 0)),
        out_shape=jax.ShapeDtypeStruct((N_POS, N_POS), jnp.int32),
    )(pos_col, pos_row)


@jax.jit
def kernel(primitives_raw, positions):
    raw01 = primitives_raw[:2]
    raw01_t = jnp.swapaxes(raw01, -1, -2)
    w = _compute_prims(raw01, raw01_t)

    pos = positions.astype(jnp.int32)
    pos_col = pos.reshape(N_POS, 1)
    pos_row = pos.reshape(1, N_POS)
    maps = _compute_maps(pos_col, w)
    steps = _compute_steps(pos_col, pos_row)
    return maps, steps
